# baseline (speedup 1.0000x reference)
import functools

import jax
import jax.numpy as jnp
from jax import lax
from jax.experimental import pallas as pl
from jax.experimental.pallas import tpu as pltpu
from jax.experimental.pallas import tpu_sc as plsc

NC, NS, L = 2, 16, 16
NW = NC * NS
R, C = 128, 32768
K = 16
HC = C // 2
BR = 32

_MESH = plsc.VectorSubcoreMesh(
    core_axis_name="c", subcore_axis_name="s", num_cores=1, num_subcores=NS
)

_KEY_BASE = 0x3E19999A


def _tc_score_body(x_ref, out_ref):
    i = pl.program_id(0)
    x = x_ref[...]
    s = jnp.sum(x, axis=1)
    q = jnp.sum(x * x, axis=1)
    var = (q - s * s * (1.0 / C)) * (1.0 / (C - 1))
    std = jnp.sqrt(jnp.maximum(var, 0.0))
    realism = jnp.where(
        std < 0.01,
        std * 10.0,
        jnp.where(std > 0.5, 0.5 / std, 1.0 - jnp.abs(std - 0.1)),
    )
    out_ref[pl.ds(i, 1)] = (realism + 0.15).reshape(1, 1, BR)


NBT = 3
RSC = R - NBT * BR

_score_tc = pl.pallas_call(
    _tc_score_body,
    grid=(NBT,),
    in_specs=[pl.BlockSpec((BR, C), lambda i: (i, 0))],
    out_specs=pl.BlockSpec((NBT, 1, BR), lambda i: (0, 0, 0)),
    out_shape=jax.ShapeDtypeStruct((NBT, 1, BR), jnp.float32),
    compiler_params=pltpu.CompilerParams(dimension_semantics=("arbitrary",)),
)


def _lane_iota():
    return lax.iota(jnp.int32, L)


_MESH2 = plsc.VectorSubcoreMesh(
    core_axis_name="c", subcore_axis_name="s", num_cores=NC, num_subcores=NS
)


@functools.partial(
    pl.kernel,
    out_type=jax.ShapeDtypeStruct((RSC, L), jnp.float32),
    mesh=_MESH2,
    scratch_types=[
        pltpu.VMEM((C,), jnp.float32),
        pltpu.VMEM((L,), jnp.float32),
        pltpu.VMEM((L,), jnp.float32),
    ],
    compiler_params=pltpu.CompilerParams(needs_layout_passes=False),
)
def _score_sc_stage(chunks_hbm, out_hbm, rowbuf, red, svmem):
    wid = lax.axis_index("s") * NC + lax.axis_index("c")
    lane = _lane_iota()
    zeros = jnp.zeros((L,), jnp.float32)

    pltpu.sync_copy(chunks_hbm.at[NBT * BR + wid], rowbuf)

    def blk(b, carry):
        s, q = carry
        off = b * (32 * L)
        x = rowbuf[pl.ds(off, L)]
        ls = x
        lq = x * x
        for i in range(1, 32):
            x = rowbuf[pl.ds(off + i * L, L)]
            ls = ls + x
            lq = lq + x * x
        return (s + ls, q + lq)

    s, q = lax.fori_loop(0, C // (32 * L), blk, (zeros, zeros))

    for shift in (8, 4, 2, 1):
        red[...] = s
        idx = lax.bitwise_xor(lane, jnp.full((L,), shift, jnp.int32))
        s = s + plsc.load_gather(red, [idx])
        red[...] = q
        q = q + plsc.load_gather(red, [idx])

    var = (q - s * s * (1.0 / C)) * (1.0 / (C - 1))
    v = jnp.maximum(var, jnp.full((L,), 1e-30, jnp.float32))
    bits = lax.bitcast_convert_type(v, jnp.int32)
    seed = jnp.full((L,), 0x5F3759DF, jnp.int32) - lax.shift_right_arithmetic(
        bits, jnp.full((L,), 1, jnp.int32)
    )
    y = lax.bitcast_convert_type(seed, jnp.float32)
    for _ in range(3):
        y = y * (1.5 - 0.5 * v * y * y)
    std = v * y
    std = 0.5 * (std + v / std)
    realism = jnp.where(
        std < 0.01,
        std * 10.0,
        jnp.where(std > 0.5, 0.5 / std, 1.0 - jnp.abs(std - 0.1)),
    )
    svmem[...] = realism + 0.15
    pltpu.sync_copy(svmem, out_hbm.at[wid])


@functools.partial(
    pl.kernel,
    out_type=(
        jax.ShapeDtypeStruct((K,), jnp.int32),
        jax.ShapeDtypeStruct((K,), jnp.float32),
    ),
    mesh=_MESH,
    scratch_types=[
        pltpu.VMEM((NBT, 1, BR), jnp.float32),
        pltpu.VMEM((RSC, L), jnp.float32),
        pltpu.VMEM((K,), jnp.int32),
        pltpu.VMEM((K,), jnp.float32),
    ],
    compiler_params=pltpu.CompilerParams(needs_layout_passes=False),
)
def _topk_stage(scores_tc_hbm, scores_sc_hbm, oidx_hbm, oscores_hbm,
                sraw, sraw_sc, tidx, tsc):
    wid = lax.axis_index("s")
    lane = _lane_iota()

    pltpu.sync_copy(scores_tc_hbm, sraw)
    pltpu.sync_copy(scores_sc_hbm, sraw_sc)

    keys = []
    for v in range(8):
        jv = lane + (16 * v)
        if v < 6:
            sv = plsc.load_gather(
                sraw,
                [
                    lax.shift_right_arithmetic(jv, jnp.full((L,), 5, jnp.int32)),
                    jnp.full((L,), 0, jnp.int32),
                    lax.bitwise_and(jv, jnp.full((L,), BR - 1, jnp.int32)),
                ],
            )
        else:
            sv = plsc.load_gather(
                sraw_sc,
                [jv - jnp.full((L,), NBT * BR, jnp.int32),
                 jnp.full((L,), 0, jnp.int32)],
            )
        bits = lax.bitcast_convert_type(sv, jnp.uint32)
        kv = lax.bitwise_or(
            lax.shift_left(bits - jnp.full((L,), _KEY_BASE, jnp.uint32),
                           jnp.full((L,), 7, jnp.uint32)),
            lax.bitcast_convert_type(jnp.full((L,), 127, jnp.int32) - jv,
                                     jnp.uint32),
        )
        ks, _ = plsc.sort_key_val(kv, kv, descending=True)
        keys.append(ks)

    def merge(ka, kb):
        kr = lax.rev(kb, (0,))
        kk = jnp.where(ka >= kr, ka, kr)
        ks, _ = plsc.sort_key_val(kk, kk, descending=True)
        return ks

    while len(keys) > 1:
        keys = [merge(keys[i], keys[i + 1]) for i in range(0, len(keys), 2)]
    top_keys = keys[0]

    @pl.when(wid == 0)
    def _():
        rows = jnp.full((L,), 127, jnp.int32) - lax.bitcast_convert_type(
            lax.bitwise_and(top_keys, jnp.full((L,), 127, jnp.uint32)),
            jnp.int32,
        )
        sbits = lax.shift_right_logical(
            top_keys, jnp.full((L,), 7, jnp.uint32)
        ) + jnp.full((L,), _KEY_BASE, jnp.uint32)
        tidx[...] = rows
        tsc[...] = lax.bitcast_convert_type(sbits, jnp.float32)
        pltpu.sync_copy(tidx, oidx_hbm)
        pltpu.sync_copy(tsc, oscores_hbm)


def _tc_gather_body(idx_ref, x_hbm, o_hbm, buf, semI, semO):
    ins = [
        pltpu.make_async_copy(
            x_hbm.at[pl.ds(idx_ref[i], 1)], buf.at[pl.ds(i, 1)], semI.at[i]
        )
        for i in range(K)
    ]
    outs = [
        pltpu.make_async_copy(buf.at[pl.ds(i, 1)], o_hbm.at[pl.ds(i, 1)], semO)
        for i in range(K)
    ]
    for cp in ins:
        cp.start()
    for i in range(K):
        ins[i].wait()
        outs[i].start()
    for cp in outs:
        cp.wait()


_gather_tc = pl.pallas_call(
    _tc_gather_body,
    grid_spec=pltpu.PrefetchScalarGridSpec(
        num_scalar_prefetch=1,
        grid=(1,),
        in_specs=[pl.BlockSpec(memory_space=pl.ANY)],
        out_specs=pl.BlockSpec(memory_space=pl.ANY),
        scratch_shapes=[
            pltpu.VMEM((K, C), jnp.float32),
            pltpu.SemaphoreType.DMA((K,)),
            pltpu.SemaphoreType.DMA,
        ],
    ),
    out_shape=jax.ShapeDtypeStruct((K, C), jnp.float32),
)


def kernel(chunks, regime_probs, keep_top_k):
    del regime_probs, keep_top_k
    scores_sc = _score_sc_stage(chunks)
    scores_tc = _score_tc(chunks)
    top_idx, top_scores = _topk_stage(scores_tc, scores_sc)
    pruned = _gather_tc(top_idx, chunks)
    return (pruned, top_scores)


# speedup vs baseline: 1.2564x; 1.2564x over previous
import functools

import jax
import jax.numpy as jnp
from jax import lax
from jax.experimental import pallas as pl
from jax.experimental.pallas import tpu as pltpu
from jax.experimental.pallas import tpu_sc as plsc

NC, NS, L = 2, 16, 16
NW = NC * NS
R, C = 128, 32768
K = 16
HC = C // 2
BR = 32

_MESH = plsc.VectorSubcoreMesh(
    core_axis_name="c", subcore_axis_name="s", num_cores=1, num_subcores=NS
)

_KEY_BASE = 0x3E19999A


def _tc_score_body(x_ref, out_ref):
    i = pl.program_id(0)
    x = x_ref[...]
    s = jnp.sum(x, axis=1)
    q = jnp.sum(x * x, axis=1)
    var = (q - s * s * (1.0 / C)) * (1.0 / (C - 1))
    std = jnp.sqrt(jnp.maximum(var, 0.0))
    realism = jnp.where(
        std < 0.01,
        std * 10.0,
        jnp.where(std > 0.5, 0.5 / std, 1.0 - jnp.abs(std - 0.1)),
    )
    out_ref[pl.ds(i, 1)] = (realism + 0.15).reshape(1, 1, BR)


_score_tc = pl.pallas_call(
    _tc_score_body,
    grid=(R // BR,),
    in_specs=[pl.BlockSpec((BR, C), lambda i: (i, 0))],
    out_specs=pl.BlockSpec((R // BR, 1, BR), lambda i: (0, 0, 0)),
    out_shape=jax.ShapeDtypeStruct((R // BR, 1, BR), jnp.float32),
    compiler_params=pltpu.CompilerParams(dimension_semantics=("arbitrary",)),
)


def _lane_iota():
    return lax.iota(jnp.int32, L)


@functools.partial(
    pl.kernel,
    out_type=(
        jax.ShapeDtypeStruct((K,), jnp.int32),
        jax.ShapeDtypeStruct((K,), jnp.float32),
    ),
    mesh=_MESH,
    scratch_types=[
        pltpu.VMEM((R // BR, 1, BR), jnp.float32),
        pltpu.VMEM((K,), jnp.int32),
        pltpu.VMEM((K,), jnp.float32),
    ],
    compiler_params=pltpu.CompilerParams(needs_layout_passes=False),
)
def _topk_stage(scores_hbm, oidx_hbm, oscores_hbm, sraw, tidx, tsc):
    wid = lax.axis_index("s")
    lane = _lane_iota()

    pltpu.sync_copy(scores_hbm, sraw)

    keys = []
    for v in range(8):
        jv = lane + (16 * v)
        sv = plsc.load_gather(
            sraw,
            [
                lax.shift_right_arithmetic(jv, jnp.full((L,), 5, jnp.int32)),
                jnp.full((L,), 0, jnp.int32),
                lax.bitwise_and(jv, jnp.full((L,), BR - 1, jnp.int32)),
            ],
        )
        bits = lax.bitcast_convert_type(sv, jnp.uint32)
        kv = lax.bitwise_or(
            lax.shift_left(bits - jnp.full((L,), _KEY_BASE, jnp.uint32),
                           jnp.full((L,), 7, jnp.uint32)),
            lax.bitcast_convert_type(jnp.full((L,), 127, jnp.int32) - jv,
                                     jnp.uint32),
        )
        ks, _ = plsc.sort_key_val(kv, kv, descending=True)
        keys.append(ks)

    def merge(ka, kb):
        kr = lax.rev(kb, (0,))
        kk = jnp.where(ka >= kr, ka, kr)
        ks, _ = plsc.sort_key_val(kk, kk, descending=True)
        return ks

    while len(keys) > 1:
        keys = [merge(keys[i], keys[i + 1]) for i in range(0, len(keys), 2)]
    top_keys = keys[0]

    @pl.when(wid == 0)
    def _():
        rows = jnp.full((L,), 127, jnp.int32) - lax.bitcast_convert_type(
            lax.bitwise_and(top_keys, jnp.full((L,), 127, jnp.uint32)),
            jnp.int32,
        )
        sbits = lax.shift_right_logical(
            top_keys, jnp.full((L,), 7, jnp.uint32)
        ) + jnp.full((L,), _KEY_BASE, jnp.uint32)
        tidx[...] = rows
        tsc[...] = lax.bitcast_convert_type(sbits, jnp.float32)
        pltpu.sync_copy(tidx, oidx_hbm)
        pltpu.sync_copy(tsc, oscores_hbm)


def _tc_gather_body(idx_ref, x_hbm, o_hbm, buf, semI, semO):
    ins = [
        pltpu.make_async_copy(
            x_hbm.at[pl.ds(idx_ref[i], 1)], buf.at[pl.ds(i, 1)], semI.at[i]
        )
        for i in range(K)
    ]
    outs = [
        pltpu.make_async_copy(buf.at[pl.ds(i, 1)], o_hbm.at[pl.ds(i, 1)], semO)
        for i in range(K)
    ]
    for cp in ins:
        cp.start()
    for i in range(K):
        ins[i].wait()
        outs[i].start()
    for cp in outs:
        cp.wait()


_gather_tc = pl.pallas_call(
    _tc_gather_body,
    grid_spec=pltpu.PrefetchScalarGridSpec(
        num_scalar_prefetch=1,
        grid=(1,),
        in_specs=[pl.BlockSpec(memory_space=pl.ANY)],
        out_specs=pl.BlockSpec(memory_space=pl.ANY),
        scratch_shapes=[
            pltpu.VMEM((K, C), jnp.float32),
            pltpu.SemaphoreType.DMA((K,)),
            pltpu.SemaphoreType.DMA,
        ],
    ),
    out_shape=jax.ShapeDtypeStruct((K, C), jnp.float32),
)


def kernel(chunks, regime_probs, keep_top_k):
    del regime_probs, keep_top_k
    scores = _score_tc(chunks)
    top_idx, top_scores = _topk_stage(scores)
    pruned = _gather_tc(top_idx, chunks)
    return (pruned, top_scores)
